# baseline (speedup 1.0000x reference)
import functools

import jax
import jax.numpy as jnp
from jax import lax
from jax.experimental import pallas as pl
from jax.experimental.pallas import tpu as pltpu
from jax.experimental.pallas import tpu_sc as plsc

NZ, NY, NX = 10, 100, 100
NVOX = NZ * NY * NX
NVOXP = 100352
XR0, XR1 = -40.0, 40.0
YR0, YR1 = -40.0, 40.0
ZR0, ZR1 = -2.0, 6.0
VS = 0.8

N = 33600
NP = 34816
NPROW = NP // 128
SLAB = NP // 16
GRP = SLAB // 128
BW = 8
NCBLK = 33
STRIPE = NVOXP // 16
ZROWS = 784
TILE = 2048


def _lerp4(v00, v01, v10, v11):
    top = v00 * 0.5 + v01 * 0.5
    bot = v10 * 0.5 + v11 * 0.5
    return top * 0.5 + bot * 0.5


def _prep_body(inp_ref, lin_ref, w_ref):
    g = inp_ref[...]
    x = _lerp4(g[0], g[1], g[2], g[3])
    y = _lerp4(g[4], g[5], g[6], g[7])
    z = _lerp4(g[8], g[9], g[10], g[11])
    cf = _lerp4(g[12], g[13], g[14], g[15])
    valid = jnp.isfinite(x) & jnp.isfinite(y) & jnp.isfinite(z)
    valid = valid & (cf > 1e-4)
    valid = valid & (x >= XR0) & (x < XR1)
    valid = valid & (y >= YR0) & (y < YR1)
    valid = valid & (z >= ZR0) & (z < ZR1)
    ix = jnp.clip(jnp.floor((x - XR0) / VS).astype(jnp.int32), 0, NX - 1)
    iy = jnp.clip(jnp.floor((y - YR0) / VS).astype(jnp.int32), 0, NY - 1)
    iz = jnp.clip(jnp.floor((z - ZR0) / VS).astype(jnp.int32), 0, NZ - 1)
    lin = iz * (NY * NX) + iy * NX + ix
    pid = (lax.broadcasted_iota(jnp.int32, (NPROW, 128), 0) * 128
           + lax.broadcasted_iota(jnp.int32, (NPROW, 128), 1))
    lin_ref[...] = jnp.where(valid, lin, pid)
    w_ref[...] = cf * valid.astype(jnp.float32)


def _weight_body(feat_ref, w_ref, fw1_ref, fw2_ref, warr_ref):
    w = w_ref[...]
    fw1_ref[...] = feat_ref[:, :128] * w
    fw2_ref[...] = feat_ref[:, 128:] * w
    warr_ref[...] = jnp.broadcast_to(w, w_ref.shape[:1] + (128,))


def _sc_scatter_body(fw1, fw2, warr, lin, vol1, vol2, ws,
                     idx_buf, upd, zbuf, acc, sem):
    c = lax.axis_index("c")
    s = lax.axis_index("s")

    z8 = jnp.zeros((1, BW), jnp.float32)

    @pl.loop(0, ZROWS)
    def _(i):
        zbuf[pl.ds(i, 1), :] = z8

    pltpu.sync_copy(lin.at[s], idx_buf)

    base = s * SLAB
    for cb in range(NCBLK):
        owner = 0 if cb < 17 else 1

        @pl.when(c == owner)
        def _(cb=cb):
            for i in range(STRIPE // ZROWS):
                pltpu.sync_copy(
                    zbuf, acc.at[pl.ds(s * STRIPE + i * ZROWS, ZROWS), :])
            plsc.subcore_barrier()

            if cb < 16:
                src = fw1.at[pl.ds(base, SLAB), pl.ds(cb * BW, BW)]
            elif cb < 32:
                src = fw2.at[pl.ds(base, SLAB), pl.ds((cb - 16) * BW, BW)]
            else:
                src = warr.at[pl.ds(base, SLAB), pl.ds(0, BW)]
            pltpu.sync_copy(src, upd)

            cps = []
            for j in range(GRP):
                cps.append(pltpu.async_copy(
                    upd.at[pl.ds(j * 128, 128), :],
                    acc.at[idx_buf.at[j]], sem, add=True))
            for cp in cps:
                cp.wait()
            plsc.subcore_barrier()

            fsrc = acc.at[pl.ds(s * STRIPE, STRIPE), :]
            if cb < 16:
                dst = vol1.at[pl.ds(s * STRIPE, STRIPE), pl.ds(cb * BW, BW)]
            elif cb < 32:
                dst = vol2.at[pl.ds(s * STRIPE, STRIPE),
                              pl.ds((cb - 16) * BW, BW)]
            else:
                dst = ws.at[pl.ds(s * STRIPE, STRIPE), pl.ds(0, BW)]
            pltpu.sync_copy(fsrc, dst)
            plsc.subcore_barrier()


def _proj_body(v1_ref, v2_ref, ws_ref, pw1_ref, pw2_ref, pb_ref, out_ref):
    wmax = jnp.maximum(ws_ref[:, 0:1], 1e-6)
    va = v1_ref[...] / wmax
    vb = v2_ref[...] / wmax
    dn = (((1,), (1,)), ((), ()))
    mm = jax.lax.dot_general(pw1_ref[...], va, dimension_numbers=dn,
                             preferred_element_type=jnp.float32)
    mm = mm + jax.lax.dot_general(pw2_ref[...], vb, dimension_numbers=dn,
                                  preferred_element_type=jnp.float32)
    out_ref[...] = mm + pb_ref[...]


@jax.jit
def _lift(inp, feat_pad, proj_w, proj_b):
    lin2, w2 = pl.pallas_call(
        _prep_body,
        out_shape=[
            jax.ShapeDtypeStruct((NPROW, 128), jnp.int32),
            jax.ShapeDtypeStruct((NPROW, 128), jnp.float32),
        ],
    )(inp)
    w_col = w2.reshape(NP)[:, None]

    wchunk = NP // 16
    fw1, fw2, warr = pl.pallas_call(
        _weight_body,
        grid=(16,),
        in_specs=[
            pl.BlockSpec((wchunk, 256), lambda i: (i, 0)),
            pl.BlockSpec((wchunk, 1), lambda i: (i, 0)),
        ],
        out_specs=[
            pl.BlockSpec((wchunk, 128), lambda i: (i, 0)),
            pl.BlockSpec((wchunk, 128), lambda i: (i, 0)),
            pl.BlockSpec((wchunk, 128), lambda i: (i, 0)),
        ],
        out_shape=[
            jax.ShapeDtypeStruct((NP, 128), jnp.float32),
            jax.ShapeDtypeStruct((NP, 128), jnp.float32),
            jax.ShapeDtypeStruct((NP, 128), jnp.float32),
        ],
    )(feat_pad, w_col)

    sc_scatter = pl.kernel(
        _sc_scatter_body,
        out_type=[
            jax.ShapeDtypeStruct((NVOXP, 128), jnp.float32),
            jax.ShapeDtypeStruct((NVOXP, 128), jnp.float32),
            jax.ShapeDtypeStruct((NVOXP, 128), jnp.float32),
        ],
        mesh=plsc.VectorSubcoreMesh(core_axis_name="c", subcore_axis_name="s",
                                    num_cores=2, num_subcores=16),
        compiler_params=pltpu.CompilerParams(use_tc_tiling_on_sc=False),
        scratch_types=[
            pltpu.VMEM((24, 128), jnp.int32),
            pltpu.VMEM((SLAB, BW), jnp.float32),
            pltpu.VMEM((ZROWS, BW), jnp.float32),
            pltpu.VMEM_SHARED((NVOXP, BW), jnp.float32),
            pltpu.SemaphoreType.DMA,
        ],
    )
    lin3 = jnp.pad(lin2.reshape(16, GRP, 128), ((0, 0), (0, 24 - GRP), (0, 0)))
    vol1, vol2, ws = sc_scatter(fw1, fw2, warr, lin3)

    out = pl.pallas_call(
        _proj_body,
        grid=(NVOXP // TILE,),
        in_specs=[
            pl.BlockSpec((TILE, 128), lambda i: (i, 0)),
            pl.BlockSpec((TILE, 128), lambda i: (i, 0)),
            pl.BlockSpec((TILE, 128), lambda i: (i, 0)),
            pl.BlockSpec((256, 128), lambda i: (0, 0)),
            pl.BlockSpec((256, 128), lambda i: (0, 0)),
            pl.BlockSpec((256, 1), lambda i: (0, 0)),
        ],
        out_specs=pl.BlockSpec((256, TILE), lambda i: (0, i)),
        out_shape=jax.ShapeDtypeStruct((256, NVOXP), jnp.float32),
    )(vol1, vol2, ws, proj_w[:, :128], proj_w[:, 128:],
      proj_b.reshape(256, 1))
    return out[:, :NVOX]


def kernel(feat_1_4, points, points_conf, proj_w, proj_b):
    b, t, v, c, h4, w4 = feat_1_4.shape
    h, w = points.shape[3], points.shape[4]
    f32 = jnp.float32

    P = points.reshape(v, h, w, 3).astype(f32)
    Cf = points_conf.reshape(v, h, w).astype(f32)

    rows = []
    taps = [(1, 1), (1, 2), (2, 1), (2, 2)]
    comps = [[], [], []]
    for (r, s) in taps:
        tp = P[:, r::4, s::4, :]
        st = tp.transpose(0, 2, 3, 1).reshape(-1, 3)
        for k in range(3):
            comps[k].append(st[:, k])
    for k in range(3):
        rows.extend(comps[k])
    for (r, s) in taps:
        rows.append(Cf[:, r::4, s::4].reshape(-1))
    inp = jnp.stack(rows)
    inp = jnp.pad(inp, ((0, 0), (0, NP - N)))
    inp = inp.reshape(16, NPROW, 128)

    feat_flat = (feat_1_4.reshape(v, c, h4, w4)
                 .transpose(0, 2, 3, 1).reshape(N, c).astype(f32))
    feat_pad = jnp.pad(feat_flat, ((0, NP - N), (0, 0)))

    out = _lift(inp, feat_pad, proj_w.astype(f32), proj_b.astype(f32))
    return out.reshape(1, 1, c, NZ, NY, NX).astype(feat_1_4.dtype)


# speedup vs baseline: 1.1767x; 1.1767x over previous
import functools

import jax
import jax.numpy as jnp
from jax import lax
from jax.experimental import pallas as pl
from jax.experimental.pallas import tpu as pltpu
from jax.experimental.pallas import tpu_sc as plsc

NZ, NY, NX = 10, 100, 100
NVOX = NZ * NY * NX
NVOXP = 100352
XR0, XR1 = -40.0, 40.0
YR0, YR1 = -40.0, 40.0
ZR0, ZR1 = -2.0, 6.0
VS = 0.8

N = 33600
NP = 34816
NPROW = NP // 128
SLAB = NP // 16
GRP = SLAB // 128
BW = 8
NCBLK = 33
STRIPE = NVOXP // 16
ZROWS = 784
TILE = NY * NX


def _lerp4(v00, v01, v10, v11):
    top = v00 * 0.5 + v01 * 0.5
    bot = v10 * 0.5 + v11 * 0.5
    return top * 0.5 + bot * 0.5


def _prep_body(inp_ref, lin_ref, w_ref):
    g = inp_ref[...]
    x, y, z, cf = g[0], g[1], g[2], g[3]
    valid = jnp.isfinite(x) & jnp.isfinite(y) & jnp.isfinite(z)
    valid = valid & (cf > 1e-4)
    valid = valid & (x >= XR0) & (x < XR1)
    valid = valid & (y >= YR0) & (y < YR1)
    valid = valid & (z >= ZR0) & (z < ZR1)
    ix = jnp.clip(jnp.floor((x - XR0) / VS).astype(jnp.int32), 0, NX - 1)
    iy = jnp.clip(jnp.floor((y - YR0) / VS).astype(jnp.int32), 0, NY - 1)
    iz = jnp.clip(jnp.floor((z - ZR0) / VS).astype(jnp.int32), 0, NZ - 1)
    lin = iz * (NY * NX) + iy * NX + ix
    pid = (lax.broadcasted_iota(jnp.int32, (NPROW, 128), 0) * 128
           + lax.broadcasted_iota(jnp.int32, (NPROW, 128), 1))
    lin_ref[...] = jnp.where(valid, lin, pid)
    w_ref[...] = cf * valid.astype(jnp.float32)


def _weight_body(feat_ref, w_ref, fw1_ref, fw2_ref, warr_ref):
    w = w_ref[...]
    fw1_ref[...] = feat_ref[:, :128] * w
    fw2_ref[...] = feat_ref[:, 128:] * w
    warr_ref[...] = jnp.broadcast_to(w, w_ref.shape[:1] + (128,))


def _sc_scatter_body(fw1, fw2, warr, lin, vol1, vol2, ws,
                     idx_buf, upd, zbuf, acc, sem):
    c = lax.axis_index("c")
    s = lax.axis_index("s")

    z8 = jnp.zeros((1, BW), jnp.float32)

    @pl.loop(0, ZROWS)
    def _(i):
        zbuf[pl.ds(i, 1), :] = z8

    pltpu.sync_copy(lin.at[s], idx_buf)

    base = s * SLAB
    for cb in range(NCBLK):
        owner = 0 if cb < 17 else 1

        @pl.when(c == owner)
        def _(cb=cb):
            for i in range(STRIPE // ZROWS):
                pltpu.sync_copy(
                    zbuf, acc.at[pl.ds(s * STRIPE + i * ZROWS, ZROWS), :])
            plsc.subcore_barrier()

            if cb < 16:
                src = fw1.at[pl.ds(base, SLAB), pl.ds(cb * BW, BW)]
            elif cb < 32:
                src = fw2.at[pl.ds(base, SLAB), pl.ds((cb - 16) * BW, BW)]
            else:
                src = warr.at[pl.ds(base, SLAB), pl.ds(0, BW)]
            pltpu.sync_copy(src, upd)

            cps = []
            for j in range(GRP):
                cps.append(pltpu.async_copy(
                    upd.at[pl.ds(j * 128, 128), :],
                    acc.at[idx_buf.at[j]], sem, add=True))
            for cp in cps:
                cp.wait()
            plsc.subcore_barrier()

            fsrc = acc.at[pl.ds(s * STRIPE, STRIPE), :]
            if cb < 16:
                dst = vol1.at[pl.ds(s * STRIPE, STRIPE), pl.ds(cb * BW, BW)]
            elif cb < 32:
                dst = vol2.at[pl.ds(s * STRIPE, STRIPE),
                              pl.ds((cb - 16) * BW, BW)]
            else:
                dst = ws.at[pl.ds(s * STRIPE, STRIPE), pl.ds(0, BW)]
            pltpu.sync_copy(fsrc, dst)
            plsc.subcore_barrier()


def _proj_body(v1_ref, v2_ref, ws_ref, pw1_ref, pw2_ref, pb_ref, out_ref):
    wmax = jnp.maximum(ws_ref[:, 0:1], 1e-6)
    va = v1_ref[...] / wmax
    vb = v2_ref[...] / wmax
    dn = (((1,), (1,)), ((), ()))
    mm = jax.lax.dot_general(pw1_ref[...], va, dimension_numbers=dn,
                             preferred_element_type=jnp.float32)
    mm = mm + jax.lax.dot_general(pw2_ref[...], vb, dimension_numbers=dn,
                                  preferred_element_type=jnp.float32)
    mm = mm + pb_ref[...]
    out_ref[...] = mm.reshape(1, 1, 128, 1, NY, NX)


@jax.jit
def _lift(inp, feat_pad, proj_w, proj_b):
    lin2, w2 = pl.pallas_call(
        _prep_body,
        out_shape=[
            jax.ShapeDtypeStruct((NPROW, 128), jnp.int32),
            jax.ShapeDtypeStruct((NPROW, 128), jnp.float32),
        ],
    )(inp)
    w_col = w2.reshape(NP)[:, None]

    wchunk = NP // 16
    fw1, fw2, warr = pl.pallas_call(
        _weight_body,
        grid=(16,),
        in_specs=[
            pl.BlockSpec((wchunk, 256), lambda i: (i, 0)),
            pl.BlockSpec((wchunk, 1), lambda i: (i, 0)),
        ],
        out_specs=[
            pl.BlockSpec((wchunk, 128), lambda i: (i, 0)),
            pl.BlockSpec((wchunk, 128), lambda i: (i, 0)),
            pl.BlockSpec((wchunk, 128), lambda i: (i, 0)),
        ],
        out_shape=[
            jax.ShapeDtypeStruct((NP, 128), jnp.float32),
            jax.ShapeDtypeStruct((NP, 128), jnp.float32),
            jax.ShapeDtypeStruct((NP, 128), jnp.float32),
        ],
    )(feat_pad, w_col)

    sc_scatter = pl.kernel(
        _sc_scatter_body,
        out_type=[
            jax.ShapeDtypeStruct((NVOXP, 128), jnp.float32),
            jax.ShapeDtypeStruct((NVOXP, 128), jnp.float32),
            jax.ShapeDtypeStruct((NVOXP, 128), jnp.float32),
        ],
        mesh=plsc.VectorSubcoreMesh(core_axis_name="c", subcore_axis_name="s",
                                    num_cores=2, num_subcores=16),
        compiler_params=pltpu.CompilerParams(use_tc_tiling_on_sc=False),
        scratch_types=[
            pltpu.VMEM((24, 128), jnp.int32),
            pltpu.VMEM((SLAB, BW), jnp.float32),
            pltpu.VMEM((ZROWS, BW), jnp.float32),
            pltpu.VMEM_SHARED((NVOXP, BW), jnp.float32),
            pltpu.SemaphoreType.DMA,
        ],
    )
    lin3 = jnp.pad(lin2.reshape(16, GRP, 128), ((0, 0), (0, 24 - GRP), (0, 0)))
    vol1, vol2, ws = sc_scatter(fw1, fw2, warr, lin3)

    out = pl.pallas_call(
        _proj_body,
        grid=(NZ, 2),
        in_specs=[
            pl.BlockSpec((TILE, 128), lambda i, j: (i, 0)),
            pl.BlockSpec((TILE, 128), lambda i, j: (i, 0)),
            pl.BlockSpec((TILE, 128), lambda i, j: (i, 0)),
            pl.BlockSpec((128, 128), lambda i, j: (j, 0)),
            pl.BlockSpec((128, 128), lambda i, j: (j, 0)),
            pl.BlockSpec((128, 1), lambda i, j: (j, 0)),
        ],
        out_specs=pl.BlockSpec(
            (1, 1, 128, 1, NY, NX),
            lambda i, j: (0, 0, j, i, 0, 0)),
        out_shape=jax.ShapeDtypeStruct((1, 1, 256, NZ, NY, NX), jnp.float32),
    )(vol1, vol2, ws, proj_w[:, :128], proj_w[:, 128:],
      proj_b.reshape(256, 1))
    return out


def kernel(feat_1_4, points, points_conf, proj_w, proj_b):
    b, t, v, c, h4, w4 = feat_1_4.shape
    h, w = points.shape[3], points.shape[4]
    f32 = jnp.float32

    P = points.reshape(v, h, w, 3).astype(f32)
    Cf = points_conf.reshape(v, h, w).astype(f32)

    def lerp(t11, t12, t21, t22):
        top = t11 * 0.5 + t12 * 0.5
        bot = t21 * 0.5 + t22 * 0.5
        return top * 0.5 + bot * 0.5

    pds = lerp(P[:, 1::4, 1::4, :], P[:, 1::4, 2::4, :],
               P[:, 2::4, 1::4, :], P[:, 2::4, 2::4, :])
    cds = lerp(Cf[:, 1::4, 1::4], Cf[:, 1::4, 2::4],
               Cf[:, 2::4, 1::4], Cf[:, 2::4, 2::4])
    sc = pds.transpose(0, 2, 3, 1).reshape(N, 3)
    rows = [sc[:, 0], sc[:, 1], sc[:, 2], cds.reshape(N)]
    inp = jnp.stack(rows)
    inp = jnp.pad(inp, ((0, 0), (0, NP - N)))
    inp = inp.reshape(4, NPROW, 128)

    feat_flat = (feat_1_4.reshape(v, c, h4, w4)
                 .transpose(0, 2, 3, 1).reshape(N, c).astype(f32))
    feat_pad = jnp.pad(feat_flat, ((0, NP - N), (0, 0)))

    out = _lift(inp, feat_pad, proj_w.astype(f32), proj_b.astype(f32))
    return out.astype(feat_1_4.dtype)
